# baseline (speedup 1.0000x reference)
import functools
from itertools import combinations

import jax
import jax.numpy as jnp
from jax import lax
from jax.experimental import pallas as pl
from jax.experimental.pallas import tpu as pltpu
from jax.experimental.pallas import tpu_sc as plsc

_NUM_CLASSES = 7
_MAX_SET_SIZE = 2
_C = 29
_L = 16


def _col_sets():
    mapping = [()]
    for set_size in range(1, _MAX_SET_SIZE + 1):
        for speakers in combinations(range(_NUM_CLASSES), set_size):
            mapping.append(speakers)
    assert len(mapping) == _C
    return [tuple(i for i, s in enumerate(mapping) if k in s)
            for k in range(_NUM_CLASSES)]


_COLS = _col_sets()


def _tree_reduce(op, xs):
    xs = list(xs)
    while len(xs) > 1:
        nxt = [op(xs[i], xs[i + 1]) for i in range(0, len(xs) - 1, 2)]
        if len(xs) % 2:
            nxt.append(xs[-1])
        xs = nxt
    return xs[0]


@functools.lru_cache(maxsize=None)
def _build_sc_call(rows):
    info = plsc.get_sparse_core_info()
    nc, ns = info.num_cores, info.num_subcores
    nw = nc * ns
    assert rows % (nw * _L) == 0
    rows_w = rows // nw
    groups = rows_w // _L
    K = _NUM_CLASSES

    mesh = plsc.VectorSubcoreMesh(core_axis_name="c", subcore_axis_name="s")

    @functools.partial(
        pl.kernel,
        out_type=jax.ShapeDtypeStruct((rows * K,), jnp.float32),
        mesh=mesh,
        scratch_types=[
            pltpu.VMEM((16,), jnp.float32),
            pltpu.VMEM((16,), jnp.float32),
        ],
        compiler_params=pltpu.CompilerParams(needs_layout_passes=False),
    )
    def sc_call(x_hbm, out_hbm, xbuf, obuf):
        wid = lax.axis_index("s") * nc + lax.axis_index("c")
        in_base = pl.multiple_of(wid * (rows_w * _C), 8)
        out_base = pl.multiple_of(wid * (rows_w * K), 8)
        pltpu.sync_copy(x_hbm.at[pl.ds(in_base, 16)], xbuf.at[pl.ds(0, 16)])

        lanes = lax.broadcasted_iota(jnp.int32, (_L,), 0)

        def group(g, carry):
            idx0 = lanes * _C + g * (_L * _C)
            vals = [plsc.load_gather(xbuf, [idx0 + c]) for c in range(_C)]
            m = _tree_reduce(jnp.maximum, vals)
            es = [jnp.exp(v - m) for v in vals]
            r = 1.0 / _tree_reduce(jnp.add, es)
            oidx0 = lanes * K + g * (_L * K)
            for k in range(K):
                ok = _tree_reduce(jnp.add, [es[c] for c in _COLS[k]]) * r
                plsc.store_scatter(obuf, [oidx0 + k], ok)
            return carry

        if True:
            pass
        else:
            lax.fori_loop(0, groups, group, 0)
        pltpu.sync_copy(obuf.at[pl.ds(0, 16)], out_hbm.at[pl.ds(out_base, 16)])

    return sc_call


def kernel(powerset, mapping_matrix):
    del mapping_matrix
    b, f, c = powerset.shape
    assert c == _C
    rows = b * f
    out_flat = _build_sc_call(rows)(powerset.reshape(-1))
    return out_flat.reshape(b, f, _NUM_CLASSES)


# speedup vs baseline: 1.4666x; 1.4666x over previous
import functools

import jax
import jax.numpy as jnp
from jax.experimental import pallas as pl
from jax.experimental.pallas import tpu as pltpu

_C = 29
_K = 7


def _body(x_ref, m_ref, o_ref):
    x = x_ref[...]
    m = jnp.max(x, axis=1, keepdims=True)
    e = jnp.exp(x - m)
    maug = jnp.concatenate(
        [m_ref[...], jnp.ones((_C, 1), jnp.float32)], axis=1)
    g = jnp.dot(e, maug, preferred_element_type=jnp.float32)
    o_ref[...] = g[:, :_K] / g[:, _K:_K + 1]


@functools.lru_cache(maxsize=None)
def _build_call(rows, block_rows):
    grid = rows // block_rows
    return pl.pallas_call(
        _body,
        grid=(grid,),
        in_specs=[
            pl.BlockSpec((block_rows, _C), lambda i: (i, 0)),
            pl.BlockSpec((_C, _K), lambda i: (0, 0)),
        ],
        out_specs=pl.BlockSpec((block_rows, _K), lambda i: (i, 0)),
        out_shape=jax.ShapeDtypeStruct((rows, _K), jnp.float32),
    )


def kernel(powerset, mapping_matrix):
    b, f, c = powerset.shape
    rows = b * f
    out = _build_call(rows, rows // 16)(powerset.reshape(rows, c),
                                        mapping_matrix)
    return out.reshape(b, f, _K)
